# baseline (speedup 1.0000x reference)
import functools

import jax
import jax.numpy as jnp
from jax import lax
from jax.experimental import pallas as pl
from jax.experimental.pallas import tpu as pltpu
from jax.experimental.pallas import tpu_sc as plsc

NUM_EMB = 1_000_000
DIM = 64
PDIM = 128
ROWS = 16384
COLS = 26
SLOTS = 32

NC = 2
NS = 16
NW = NC * NS

L = 16

_mesh = plsc.VectorSubcoreMesh(core_axis_name="c", subcore_axis_name="s")

CB = 4
R_PER_W = ROWS // NW
N_CHUNKS = R_PER_W // CB
CROWS = CB * SLOTS

assert R_PER_W % CB == 0 and N_CHUNKS >= 6
N_LOOP = (N_CHUNKS - 4) // 3


@functools.partial(
    pl.kernel,
    mesh=_mesh,
    out_type=jax.ShapeDtypeStruct((ROWS * SLOTS, PDIM), jnp.float32),
    scratch_types=[
        pltpu.VMEM((R_PER_W, COLS), jnp.int32),
        pltpu.VMEM((CROWS, PDIM), jnp.float32),
        pltpu.VMEM((CROWS, PDIM), jnp.float32),
        pltpu.VMEM((CROWS, PDIM), jnp.float32),
        pltpu.SemaphoreType.DMA,
        pltpu.SemaphoreType.DMA,
        pltpu.SemaphoreType.DMA,
        pltpu.SemaphoreType.DMA,
        pltpu.SemaphoreType.DMA,
        pltpu.SemaphoreType.DMA,
    ],
)
def _emb_lookup(idx_hbm, table_hbm, out_hbm, idx_v, r0, r1, r2,
                g0, g1, g2, o0, o1, o2):
    wid = lax.axis_index("s") * NC + lax.axis_index("c")
    row0 = wid * R_PER_W
    rbufs = (r0, r1, r2)
    gsems = (g0, g1, g2)
    osems = (o0, o1, o2)

    def gather(i, b):
        for blk in range(CB):
            pltpu.async_copy(
                table_hbm.at[idx_v.at[i * CB + blk]],
                rbufs[b].at[pl.ds(blk * SLOTS, COLS)],
                gsems[b],
            )

    def wait_gather(b):
        for blk in range(CB):
            pltpu.make_async_copy(
                table_hbm.at[idx_v.at[blk]],
                rbufs[b].at[pl.ds(blk * SLOTS, COLS)],
                gsems[b],
            ).wait()

    def store(i, b):
        pltpu.async_copy(
            rbufs[b], out_hbm.at[pl.ds((row0 + i * CB) * SLOTS, CROWS)],
            osems[b])

    def wait_store(b):
        pltpu.make_async_copy(
            rbufs[b], out_hbm.at[pl.ds(0, CROWS)], osems[b]).wait()

    pltpu.sync_copy(idx_hbm.at[pl.ds(row0, R_PER_W)], idx_v)

    gather(0, 0)
    gather(1, 1)
    wait_gather(0)
    store(0, 0)
    gather(2, 2)
    wait_gather(1)
    store(1, 1)
    wait_store(0)
    gather(3, 0)

    def body(K, _):
        for a in range(3):
            i = 3 * K + 2 + a
            b = (2 + a) % 3
            wait_gather(b)
            store(i, b)
            wait_store((b + 2) % 3)
            gather(i + 2, (b + 2) % 3)
        return 0

    lax.fori_loop(0, N_LOOP, body, 0)

    first = 3 * N_LOOP + 2
    for i in range(first, N_CHUNKS):
        b = i % 3
        wait_gather(b)
        store(i, b)
        if i + 2 <= N_CHUNKS - 1:
            wait_store((b + 2) % 3)
            gather(i + 2, (b + 2) % 3)
    for i in range(N_CHUNKS - 3, N_CHUNKS):
        wait_store(i % 3)


def kernel(x, Weights):
    table = jnp.pad(Weights, ((0, 0), (0, PDIM - DIM)))
    out = _emb_lookup(x.astype(jnp.int32), table)
    return out.reshape(ROWS, SLOTS, PDIM)[:, :COLS, :DIM]


# speedup vs baseline: 1.0021x; 1.0021x over previous
import functools

import jax
import jax.numpy as jnp
from jax import lax
from jax.experimental import pallas as pl
from jax.experimental.pallas import tpu as pltpu
from jax.experimental.pallas import tpu_sc as plsc

NUM_EMB = 1_000_000
DIM = 64
PDIM = 128
ROWS = 16384
COLS = 26
SLOTS = 32

NC = 2
NS = 16
NW = NC * NS

L = 16

_mesh = plsc.VectorSubcoreMesh(core_axis_name="c", subcore_axis_name="s")

CB = 8
R_PER_W = ROWS // NW
N_CHUNKS = R_PER_W // CB
CROWS = CB * SLOTS

assert R_PER_W % CB == 0 and N_CHUNKS % 2 == 0


@functools.partial(
    pl.kernel,
    mesh=_mesh,
    out_type=jax.ShapeDtypeStruct((ROWS * SLOTS, PDIM), jnp.float32),
    scratch_types=[
        pltpu.VMEM((R_PER_W, COLS), jnp.int32),
        pltpu.VMEM((CROWS, PDIM), jnp.float32),
        pltpu.VMEM((CROWS, PDIM), jnp.float32),
        pltpu.SemaphoreType.DMA,
        pltpu.SemaphoreType.DMA,
        pltpu.SemaphoreType.DMA,
    ],
)
def _emb_lookup(idx_hbm, table_hbm, out_hbm, idx_v, rows0, rows1, gsem,
                osem0, osem1):
    wid = lax.axis_index("s") * NC + lax.axis_index("c")
    row0 = wid * R_PER_W

    def gather(i, rbuf):
        for blk in range(CB):
            pltpu.async_copy(
                table_hbm.at[idx_v.at[i * CB + blk]],
                rbuf.at[pl.ds(blk * SLOTS, COLS)],
                gsem,
            )

    def wait_gather(rbuf):
        for blk in range(CB):
            pltpu.make_async_copy(
                table_hbm.at[idx_v.at[blk]],
                rbuf.at[pl.ds(blk * SLOTS, COLS)],
                gsem,
            ).wait()

    def store(i, rbuf, osem):
        pltpu.async_copy(
            rbuf, out_hbm.at[pl.ds((row0 + i * CB) * SLOTS, CROWS)], osem)

    def wait_store(rbuf, osem):
        pltpu.make_async_copy(
            rbuf, out_hbm.at[pl.ds(0, CROWS)], osem).wait()

    pltpu.sync_copy(idx_hbm.at[pl.ds(row0, R_PER_W)], idx_v)

    gather(0, rows0)
    wait_gather(rows0)
    gather(1, rows1)
    store(0, rows0, osem0)

    def body(k, _):
        i1 = 2 * k + 1
        wait_gather(rows1)
        wait_store(rows0, osem0)
        gather(i1 + 1, rows0)
        store(i1, rows1, osem1)
        i2 = 2 * k + 2
        wait_gather(rows0)
        wait_store(rows1, osem1)
        gather(i2 + 1, rows1)
        store(i2, rows0, osem0)
        return 0

    lax.fori_loop(0, N_CHUNKS // 2 - 1, body, 0)

    wait_gather(rows1)
    wait_store(rows0, osem0)
    store(N_CHUNKS - 1, rows1, osem1)
    wait_store(rows1, osem1)


def kernel(x, Weights):
    table = jnp.pad(Weights, ((0, 0), (0, PDIM - DIM)))
    out = _emb_lookup(x.astype(jnp.int32), table)
    return out.reshape(ROWS, SLOTS, PDIM)[:, :COLS, :DIM]
